# baseline (speedup 1.0000x reference)
import jax
import jax.numpy as jnp
from jax.experimental import pallas as pl
from jax.experimental.pallas import tpu as pltpu

_NL = 8
_SIG = 0.5


def _body(pa_ref, km_ref, kla_ref, rla_ref,
          pb_ref, rm_ref, klb_ref, rlb_ref,
          loss_ref, tab_ref):
    b = pl.program_id(0)
    nb = pl.num_programs(0)

    tab = tab_ref[...]
    klb = klb_ref[0, 0]
    rlb = rlb_ref[0, 0]
    rm = rm_ref[0, 0]
    H = klb.shape[0]

    inv_k = 1.0 / (tab[0:1] + 1.0)
    g_t = [tab[1 + c:2 + c] * inv_k for c in range(4)]
    lane = jax.lax.broadcasted_iota(jnp.int32, (1, _NL), 1)
    rinv_t = jnp.where(lane > 0, 1.0 / (tab[5:6] + 1.0), 1.0)

    def gather(t, idx):
        tb = jnp.broadcast_to(t, (H, _NL))
        return jnp.take_along_axis(tb, idx, axis=1, mode="promise_in_bounds")

    acc = jnp.zeros_like(rm)
    for c in range(4):
        fp = pb_ref[0, c] * rm
        d = fp - gather(g_t[c], klb)
        acc = acc + d * d
    dd = jnp.maximum(jnp.sqrt(acc) - _SIG, 0.0)
    dd = jnp.log(dd * dd + 1.0)
    s = jnp.sum(dd * gather(rinv_t, rlb))

    numk = jnp.max(klb).astype(jnp.float32)
    is_first = b == 1
    is_last = b == nb - 1
    prev = jnp.where(is_first, 0.0, loss_ref[0, 0])
    tot = prev + s
    loss_ref[0, 0] = jnp.where(is_last, tot / numk, tot)

    kla = kla_ref[0, 0]
    rla = rla_ref[0, 0]
    km = km_ref[0, 0]
    z11 = jnp.zeros((1, 1), jnp.float32)

    def msum(mask, data):
        return jnp.sum(jnp.where(mask, data, 0.0), axis=(0, 1), keepdims=True)

    kmask = [kla == l for l in range(1, _NL)]
    rmask = [rla == l for l in range(1, _NL)]
    ks_t = jnp.concatenate([z11] + [msum(m, km) for m in kmask], axis=1)
    rs_t = jnp.concatenate([z11] + [msum(m, km) for m in rmask], axis=1)
    cs_t = [jnp.concatenate([z11] + [msum(m, pa_ref[0, c]) for m in kmask],
                            axis=1) for c in range(4)]
    tab_ref[...] = jnp.concatenate([ks_t] + cs_t + [rs_t], axis=0)


def kernel(pred_similarities, regions_mask, kernels_mask, text_mask_ndi_labels, kernel_mask_ndi_labels):
    B, C, H, W = pred_similarities.shape

    def a_idx(b):
        return (jnp.minimum(b, B - 1), 0, 0, 0)

    def b_idx(b):
        return (jnp.maximum(b - 1, 0), 0, 0, 0)

    img_a = lambda: pl.BlockSpec((1, 1, H, W), a_idx)
    img_b = lambda: pl.BlockSpec((1, 1, H, W), b_idx)

    loss = pl.pallas_call(
        _body,
        grid=(B + 1,),
        in_specs=[
            pl.BlockSpec((1, C, H, W), a_idx),
            img_a(),
            img_a(),
            img_a(),
            pl.BlockSpec((1, C, H, W), b_idx),
            img_b(),
            img_b(),
            img_b(),
        ],
        out_specs=pl.BlockSpec(memory_space=pltpu.SMEM),
        out_shape=jax.ShapeDtypeStruct((1, 1), jnp.float32),
        scratch_shapes=[pltpu.VMEM((6, _NL), jnp.float32)],
    )(pred_similarities, kernels_mask, kernel_mask_ndi_labels, text_mask_ndi_labels,
      pred_similarities, regions_mask, kernel_mask_ndi_labels, text_mask_ndi_labels)

    return loss[0, 0]
